# Initial kernel scaffold; baseline (speedup 1.0000x reference)
import jax
import jax.numpy as jnp
from jax.experimental import pallas as pl


def kernel(adj, x, W0, b0, W1, b1):
    raise NotImplementedError("write your pallas kernel here")


# speedup vs baseline: 3.3003x; 3.3003x over previous
import functools

import jax
import jax.numpy as jnp
from jax import lax
from jax.experimental import pallas as pl
from jax.experimental.pallas import tpu as pltpu
from jax.experimental.pallas import tpu_sc as plsc

N = 10000
D = 128
E = 320000

NC = 2
NS = 16
NW = NC * NS

CHUNK = 128
CPT = 80
TOTCH = NW * CPT
EPAD = TOTCH * CHUNK

NACC = 10240
ZR = NACC // NS
WR = 624


def _spmm_sc(support, src2d, dst2d):
  mesh = plsc.VectorSubcoreMesh(core_axis_name="c", subcore_axis_name="s")

  @functools.partial(
      pl.kernel,
      mesh=mesh,
      out_type=jax.ShapeDtypeStruct((NC, N, D), jnp.float32),
      scratch_types=[
          pltpu.VMEM((CPT, CHUNK), jnp.int32),
          pltpu.VMEM((CPT, CHUNK), jnp.int32),
          pltpu.VMEM((CHUNK, D), jnp.float32),
          pltpu.VMEM_SHARED((NACC, D), jnp.float32),
          pltpu.SemaphoreType.DMA,
      ],
  )
  def k(sup_hbm, src_hbm, dst_hbm, out_hbm, srcv, dstv, gbuf, acc, sem):
    c = lax.axis_index("c")
    s = lax.axis_index("s")
    wid = s * NC + c

    zero = jnp.zeros((16,), jnp.float32)

    def zbody(i, carry):
      gbuf[i // (D // 16), pl.ds((i % (D // 16)) * 16, 16)] = zero
      return carry

    lax.fori_loop(0, CHUNK * (D // 16), zbody, 0)

    zbase = s * ZR
    for t in range(ZR // CHUNK):
      pltpu.sync_copy(gbuf, acc.at[pl.ds(zbase + t * CHUNK, CHUNK)])
    plsc.subcore_barrier()

    cbase = wid * CPT
    pltpu.sync_copy(src_hbm.at[pl.ds(cbase, CPT)], srcv)
    pltpu.sync_copy(dst_hbm.at[pl.ds(cbase, CPT)], dstv)

    def body(j, carry):
      pltpu.async_copy(sup_hbm.at[srcv.at[j]], gbuf, sem).wait()
      pltpu.sync_copy(gbuf, acc.at[dstv.at[j]], add=True)
      return carry

    lax.fori_loop(0, CPT, body, 0)
    plsc.subcore_barrier()

    wbase = s * WR
    pltpu.sync_copy(acc.at[pl.ds(wbase, WR)], out_hbm.at[c, pl.ds(wbase, WR)])

    @pl.when(s == NS - 1)
    def _tail():
      pltpu.sync_copy(acc.at[pl.ds(NS * WR, N - NS * WR)],
                      out_hbm.at[c, pl.ds(NS * WR, N - NS * WR)])

  return k(support, src2d, dst2d)


def _mm(x, w):
  bm = 1000

  def body(x_ref, w_ref, o_ref):
    o_ref[...] = jnp.dot(x_ref[...], w_ref[...],
                         preferred_element_type=jnp.float32)

  return pl.pallas_call(
      body,
      grid=(x.shape[0] // bm,),
      in_specs=[
          pl.BlockSpec((bm, D), lambda i: (i, 0)),
          pl.BlockSpec((D, D), lambda i: (0, 0)),
      ],
      out_specs=pl.BlockSpec((bm, D), lambda i: (i, 0)),
      out_shape=jax.ShapeDtypeStruct((x.shape[0], D), jnp.float32),
  )(x, w)


def _combine_mm(pa, pb, b, w):
  bm = 1000

  def body(pa_ref, pb_ref, b_ref, w_ref, o_ref):
    h = pa_ref[...] + pb_ref[...] + b_ref[...]
    o_ref[...] = jnp.dot(h, w_ref[...], preferred_element_type=jnp.float32)

  return pl.pallas_call(
      body,
      grid=(N // bm,),
      in_specs=[
          pl.BlockSpec((bm, D), lambda i: (i, 0)),
          pl.BlockSpec((bm, D), lambda i: (i, 0)),
          pl.BlockSpec((1, D), lambda i: (0, 0)),
          pl.BlockSpec((D, D), lambda i: (0, 0)),
      ],
      out_specs=pl.BlockSpec((bm, D), lambda i: (i, 0)),
      out_shape=jax.ShapeDtypeStruct((N, D), jnp.float32),
  )(pa, pb, b.reshape(1, D), w)


def _combine_logsoftmax(pa, pb, b):
  bm = 1000

  def body(pa_ref, pb_ref, b_ref, o_ref):
    h = pa_ref[...] + pb_ref[...] + b_ref[...]
    m = jnp.max(h, axis=1, keepdims=True)
    e = jnp.exp(h - m)
    ssum = jnp.sum(e, axis=1, keepdims=True)
    o_ref[...] = h - m - jnp.log(ssum)

  return pl.pallas_call(
      body,
      grid=(N // bm,),
      in_specs=[
          pl.BlockSpec((bm, D), lambda i: (i, 0)),
          pl.BlockSpec((bm, D), lambda i: (i, 0)),
          pl.BlockSpec((1, D), lambda i: (0, 0)),
      ],
      out_specs=pl.BlockSpec((bm, D), lambda i: (i, 0)),
      out_shape=jax.ShapeDtypeStruct((N, D), jnp.float32),
  )(pa, pb, b.reshape(1, D))


def kernel(adj, x, W0, b0, W1, b1):
  src = adj[0]
  dst = adj[1]
  pad = EPAD - E
  src2d = jnp.concatenate([src, jnp.zeros((pad,), jnp.int32)]).reshape(
      TOTCH, CHUNK)
  dst2d = jnp.concatenate([dst, jnp.full((pad,), N, jnp.int32)]).reshape(
      TOTCH, CHUNK)

  sup0 = _mm(x, W0)
  p = _spmm_sc(sup0, src2d, dst2d)
  sup1 = _combine_mm(p[0], p[1], b0, W1)
  q = _spmm_sc(sup1, src2d, dst2d)
  return _combine_logsoftmax(q[0], q[1], b1)


# speedup vs baseline: 4.9280x; 1.4932x over previous
import functools

import jax
import jax.numpy as jnp
from jax import lax
from jax.experimental import pallas as pl
from jax.experimental.pallas import tpu as pltpu
from jax.experimental.pallas import tpu_sc as plsc

N = 10000
D = 128
DH = D // 2
E = 320000

NC = 2
NS = 16

CHUNK = 128
CPT = 160
TOTCH = NS * CPT
EPAD = TOTCH * CHUNK

NBUF = 4
NACC = 10112
ZR = NACC // NS
WR = 624


def _spmm_sc(sup2, src2d, dst2d):
  mesh = plsc.VectorSubcoreMesh(core_axis_name="c", subcore_axis_name="s")

  @functools.partial(
      pl.kernel,
      mesh=mesh,
      compiler_params=pltpu.CompilerParams(use_tc_tiling_on_sc=False),
      out_type=jax.ShapeDtypeStruct((NC, N, DH), jnp.float32),
      scratch_types=[
          pltpu.VMEM((CPT, CHUNK), jnp.int32),
          pltpu.VMEM((CPT, CHUNK), jnp.int32),
          pltpu.VMEM((NBUF, CHUNK, DH), jnp.float32),
          pltpu.VMEM_SHARED((NACC, DH), jnp.float32),
      ] + [pltpu.SemaphoreType.DMA] * NBUF,
  )
  def k(sup_hbm, src_hbm, dst_hbm, out_hbm, srcv, dstv, gbuf, acc, *sems):
    c = lax.axis_index("c")
    s = lax.axis_index("s")

    zero = jnp.zeros((16,), jnp.float32)

    def zbody(i, carry):
      gbuf[0, i // (DH // 16), pl.ds((i % (DH // 16)) * 16, 16)] = zero
      return carry

    lax.fori_loop(0, CHUNK * (DH // 16), zbody, 0)

    zbase = s * ZR
    for t in range(ZR // CHUNK):
      pltpu.sync_copy(gbuf.at[0], acc.at[pl.ds(zbase + t * CHUNK, CHUNK)])
    zrem = ZR - (ZR // CHUNK) * CHUNK
    if zrem:
      pltpu.sync_copy(gbuf.at[0, pl.ds(0, zrem)],
                      acc.at[pl.ds(zbase + ZR - zrem, zrem)])
    plsc.subcore_barrier()

    cbase = s * CPT
    pltpu.sync_copy(src_hbm.at[pl.ds(cbase, CPT)], srcv)
    pltpu.sync_copy(dst_hbm.at[pl.ds(cbase, CPT)], dstv)

    for b in range(NBUF):
      pltpu.async_copy(sup_hbm.at[c].at[srcv.at[b]], gbuf.at[b], sems[b])

    def body(g, carry):
      j = g * NBUF
      for b in range(NBUF):
        jj = j + b
        pltpu.make_async_copy(
            sup_hbm.at[c].at[srcv.at[jj]], gbuf.at[b], sems[b]).wait()
        pltpu.sync_copy(gbuf.at[b], acc.at[dstv.at[jj]], add=True)

        @pl.when(jj + NBUF < CPT)
        def _next():
          pltpu.async_copy(
              sup_hbm.at[c].at[srcv.at[jj + NBUF]], gbuf.at[b], sems[b])

      return carry

    lax.fori_loop(0, CPT // NBUF, body, 0)
    plsc.subcore_barrier()

    wbase = s * WR
    pltpu.sync_copy(acc.at[pl.ds(wbase, WR)], out_hbm.at[c, pl.ds(wbase, WR)])

    @pl.when(s == NS - 1)
    def _tail():
      pltpu.sync_copy(acc.at[pl.ds(NS * WR, N - NS * WR)],
                      out_hbm.at[c, pl.ds(NS * WR, N - NS * WR)])

  return k(sup2, src2d, dst2d)


_BM = 1000


def _mm_split(x, w2):
  def body(x_ref, w_ref, o_ref):
    o_ref[0, ...] = jnp.dot(x_ref[...], w_ref[0, ...],
                            preferred_element_type=jnp.float32)

  return pl.pallas_call(
      body,
      grid=(2, N // _BM),
      in_specs=[
          pl.BlockSpec((_BM, D), lambda j, i: (i, 0)),
          pl.BlockSpec((1, D, DH), lambda j, i: (j, 0, 0)),
      ],
      out_specs=pl.BlockSpec((1, _BM, DH), lambda j, i: (j, i, 0)),
      out_shape=jax.ShapeDtypeStruct((2, N, DH), jnp.float32),
  )(x, w2)


def _combine_mm_split(h2c, b, w2):
  def body(ha_ref, hb_ref, b_ref, w_ref, o_ref):
    ha = ha_ref[0, ...] + b_ref[:, :DH]
    hb = hb_ref[0, ...] + b_ref[:, DH:]
    o_ref[0, ...] = (
        jnp.dot(ha, w_ref[0, :DH, :], preferred_element_type=jnp.float32)
        + jnp.dot(hb, w_ref[0, DH:, :], preferred_element_type=jnp.float32))

  return pl.pallas_call(
      body,
      grid=(2, N // _BM),
      in_specs=[
          pl.BlockSpec((1, _BM, DH), lambda j, i: (0, i, 0)),
          pl.BlockSpec((1, _BM, DH), lambda j, i: (1, i, 0)),
          pl.BlockSpec((1, D), lambda j, i: (0, 0)),
          pl.BlockSpec((1, D, DH), lambda j, i: (j, 0, 0)),
      ],
      out_specs=pl.BlockSpec((1, _BM, DH), lambda j, i: (j, i, 0)),
      out_shape=jax.ShapeDtypeStruct((2, N, DH), jnp.float32),
  )(h2c, h2c, b.reshape(1, D), w2)


def _combine_logsoftmax(q2c, b):
  def body(qa_ref, qb_ref, b_ref, o_ref):
    h = jnp.concatenate([qa_ref[0, ...], qb_ref[0, ...]], axis=1) + b_ref[...]
    m = jnp.max(h, axis=1, keepdims=True)
    e = jnp.exp(h - m)
    ssum = jnp.sum(e, axis=1, keepdims=True)
    o_ref[...] = h - m - jnp.log(ssum)

  return pl.pallas_call(
      body,
      grid=(N // _BM,),
      in_specs=[
          pl.BlockSpec((1, _BM, DH), lambda i: (0, i, 0)),
          pl.BlockSpec((1, _BM, DH), lambda i: (1, i, 0)),
          pl.BlockSpec((1, D), lambda i: (0, 0)),
      ],
      out_specs=pl.BlockSpec((_BM, D), lambda i: (i, 0)),
      out_shape=jax.ShapeDtypeStruct((N, D), jnp.float32),
  )(q2c, q2c, b.reshape(1, D))


def kernel(adj, x, W0, b0, W1, b1):
  src = adj[0]
  dst = adj[1]
  pad = EPAD - E
  src2d = jnp.concatenate([src, jnp.zeros((pad,), jnp.int32)]).reshape(
      TOTCH, CHUNK)
  dst2d = jnp.concatenate([dst, jnp.full((pad,), N, jnp.int32)]).reshape(
      TOTCH, CHUNK)

  W0s = jnp.stack([W0[:, :DH], W0[:, DH:]])
  W1s = jnp.stack([W1[:, :DH], W1[:, DH:]])
  sup0 = _mm_split(x, W0s)
  p = _spmm_sc(sup0, src2d, dst2d)
  sup1 = _combine_mm_split(p, b0, W1s)
  q = _spmm_sc(sup1, src2d, dst2d)
  return _combine_logsoftmax(q, b1)
